# baseline (speedup 1.0000x reference)
import functools

import jax
import jax.numpy as jnp
from jax import lax
from jax.experimental import pallas as pl
from jax.experimental.pallas import tpu as pltpu
from jax.experimental.pallas import tpu_sc as plsc

NC = 2
NS = 16
NW = NC * NS
L = 16


def _sc_gather_sum(cs_flat, em_flat, an_w, ae_w, B, N, D, K, CH, C5):
    NPB = NW // B
    LAST = N - (NPB - 1) * CH

    mesh = plsc.VectorSubcoreMesh(core_axis_name="c", subcore_axis_name="s")

    @functools.partial(
        pl.kernel,
        out_type=jax.ShapeDtypeStruct((B, N, D), jnp.float32),
        mesh=mesh,
        scratch_types=[
            pltpu.VMEM((K, C5, 128), jnp.int32),
            pltpu.VMEM((K, C5, 128), jnp.int32),
            pltpu.VMEM((CH, D), jnp.float32),
            pltpu.SemaphoreType.DMA,
        ],
    )
    def body(cs_ref, em_ref, an_ref, ae_ref, out_ref, idxn_v, idxe_v, acc_v, sem):
        wid = lax.axis_index("s") * NC + lax.axis_index("c")
        b = wid // NPB
        j = wid - b * NPB

        pltpu.sync_copy(an_ref.at[wid], idxn_v)
        pltpu.sync_copy(ae_ref.at[wid], idxe_v)

        init = [
            pltpu.async_copy(
                cs_ref.at[idxn_v.at[0, c]], acc_v.at[pl.ds(c * 128, 128)], sem
            )
            for c in range(C5)
        ]
        for d in init:
            d.wait()

        descs = []
        for g in range(1, K):
            for c in range(C5):
                descs.append(
                    pltpu.async_copy(
                        cs_ref.at[idxn_v.at[g, c]],
                        acc_v.at[pl.ds(c * 128, 128)],
                        sem,
                        add=True,
                    )
                )
        for g in range(K):
            for c in range(C5):
                descs.append(
                    pltpu.async_copy(
                        em_ref.at[idxe_v.at[g, c]],
                        acc_v.at[pl.ds(c * 128, 128)],
                        sem,
                        add=True,
                    )
                )
        for d in descs:
            d.wait()

        row0 = pl.multiple_of(j * CH, 8)

        @pl.when(j == NPB - 1)
        def _():
            pltpu.sync_copy(acc_v.at[pl.ds(0, LAST)], out_ref.at[b, pl.ds(row0, LAST)])

        @pl.when(j != NPB - 1)
        def _():
            pltpu.sync_copy(acc_v, out_ref.at[b, pl.ds(row0, CH)])

    return body(cs_flat, em_flat, an_w, ae_w)


def _mlp_block(ns_ref, cs_ref, cn_ref, ce_ref, cs0_ref, em0_ref, wn_ref, wc_ref,
               b_ref, out_ref):
    ns = ns_ref[...] - cn_ref[...] * cs0_ref[0] - ce_ref[...] * em0_ref[0]
    acc = jnp.dot(ns, wn_ref[...], preferred_element_type=jnp.float32)
    acc += jnp.dot(cs_ref[...], wc_ref[...], preferred_element_type=jnp.float32)
    out_ref[0] = jnp.tanh(acc + b_ref[...])


def kernel(current_state, edges_m, A_nodes, A_edges, W, b):
    B, N, D = current_state.shape
    E = edges_m.shape[1]
    K = A_nodes.shape[-1]

    NPB = NW // B
    CH = -(-N // NPB)
    CH = -(-CH // 128) * 128
    C5 = CH // 128
    NPAD = NPB * CH

    cs_flat = current_state.reshape(B * N, D)
    em_flat = edges_m.reshape(B * E, D)
    cntn = jnp.sum(A_nodes == 0, axis=-1, dtype=jnp.float32).reshape(B * N, 1)
    cnte = jnp.sum(A_edges == 0, axis=-1, dtype=jnp.float32).reshape(B * N, 1)

    def prep_idx(A, stride):
        idx = A.astype(jnp.int32) + (jnp.arange(B, dtype=jnp.int32) * stride)[:, None, None]
        idx_t = idx.transpose(0, 2, 1)
        pad = (jnp.arange(NPAD - N, dtype=jnp.int32) * 1009) % (B * stride)
        pad = jnp.broadcast_to(pad[None, None, :], (B, K, NPAD - N))
        idx_t = jnp.concatenate([idx_t, pad], axis=2)
        idx_t = idx_t.reshape(B, K, NPB, C5, 128).transpose(0, 2, 1, 3, 4)
        return idx_t.reshape(NW, K, C5, 128)

    an_w = prep_idx(A_nodes, N)
    ae_w = prep_idx(A_edges, E)

    ns = _sc_gather_sum(cs_flat, em_flat, an_w, ae_w, B, N, D, K, CH, C5)

    BLK = 1000
    wn = W[:, :D].T
    wc = W[:, D:].T
    ns2 = ns.reshape(B * N, D)
    cs2 = current_state.reshape(B * N, D)
    cs0 = current_state[:, 0:1, :]
    em0 = edges_m[:, 0:1, :]
    nb = N // BLK
    out = pl.pallas_call(
        _mlp_block,
        grid=(B * N // BLK,),
        in_specs=[
            pl.BlockSpec((BLK, D), lambda i: (i, 0)),
            pl.BlockSpec((BLK, D), lambda i: (i, 0)),
            pl.BlockSpec((BLK, 1), lambda i: (i, 0)),
            pl.BlockSpec((BLK, 1), lambda i: (i, 0)),
            pl.BlockSpec((1, 1, D), lambda i: (i // nb, 0, 0)),
            pl.BlockSpec((1, 1, D), lambda i: (i // nb, 0, 0)),
            pl.BlockSpec((D, D), lambda i: (0, 0)),
            pl.BlockSpec((D, D), lambda i: (0, 0)),
            pl.BlockSpec((1, D), lambda i: (0, 0)),
        ],
        out_specs=pl.BlockSpec((1, BLK, D), lambda i: (i // nb, i % nb, 0)),
        out_shape=jax.ShapeDtypeStruct((B, N, D), jnp.float32),
    )(ns2, cs2, cntn, cnte, cs0, em0, wn, wc, b.reshape(1, D))
    return out


# speedup vs baseline: 1.0323x; 1.0323x over previous
import functools

import jax
import jax.numpy as jnp
from jax import lax
from jax.experimental import pallas as pl
from jax.experimental.pallas import tpu as pltpu
from jax.experimental.pallas import tpu_sc as plsc

NC = 2
NS = 16
NW = NC * NS
L = 16


def _sc_gather_sum(cs_flat, em_flat, an_w, ae_w, B, N, D, K, CH, C5):
    NPB = NW // B
    LAST = N - (NPB - 1) * CH

    mesh = plsc.VectorSubcoreMesh(core_axis_name="c", subcore_axis_name="s")

    @functools.partial(
        pl.kernel,
        out_type=jax.ShapeDtypeStruct((B, N, D), jnp.float32),
        mesh=mesh,
        scratch_types=[
            pltpu.VMEM((K, C5, 128), jnp.int32),
            pltpu.VMEM((K, C5, 128), jnp.int32),
            pltpu.VMEM((CH, D), jnp.float32),
            pltpu.SemaphoreType.DMA,
        ],
    )
    def body(cs_ref, em_ref, an_ref, ae_ref, out_ref, idxn_v, idxe_v, acc_v, sem):
        wid = lax.axis_index("s") * NC + lax.axis_index("c")
        b = wid // NPB
        j = wid - b * NPB

        pltpu.sync_copy(an_ref.at[wid], idxn_v)
        pltpu.sync_copy(ae_ref.at[wid], idxe_v)

        init = [
            pltpu.async_copy(
                cs_ref.at[idxn_v.at[0, c]], acc_v.at[pl.ds(c * 128, 128)], sem
            )
            for c in range(C5)
        ]
        for d in init:
            d.wait()

        descs = []
        for g in range(1, K):
            for c in range(C5):
                descs.append(
                    pltpu.async_copy(
                        cs_ref.at[idxn_v.at[g, c]],
                        acc_v.at[pl.ds(c * 128, 128)],
                        sem,
                        add=True,
                    )
                )
        for g in range(K):
            for c in range(C5):
                descs.append(
                    pltpu.async_copy(
                        em_ref.at[idxe_v.at[g, c]],
                        acc_v.at[pl.ds(c * 128, 128)],
                        sem,
                        add=True,
                    )
                )
        for d in descs:
            d.wait()

        row0 = pl.multiple_of(j * CH, 8)

        @pl.when(j == NPB - 1)
        def _():
            pltpu.sync_copy(acc_v.at[pl.ds(0, LAST)], out_ref.at[b, pl.ds(row0, LAST)])

        @pl.when(j != NPB - 1)
        def _():
            pltpu.sync_copy(acc_v, out_ref.at[b, pl.ds(row0, CH)])

    return body(cs_flat, em_flat, an_w, ae_w)


def _mlp_block(ns_ref, cs_ref, cn_ref, ce_ref, cs0_ref, em0_ref, wn_ref, wc_ref,
               b_ref, out_ref):
    ns = ns_ref[...] - cn_ref[...] * cs0_ref[0] - ce_ref[...] * em0_ref[0]
    acc = jnp.dot(ns, wn_ref[...], preferred_element_type=jnp.float32)
    acc += jnp.dot(cs_ref[...], wc_ref[...], preferred_element_type=jnp.float32)
    out_ref[0] = jnp.tanh(acc + b_ref[...])


def kernel(current_state, edges_m, A_nodes, A_edges, W, b):
    B, N, D = current_state.shape
    E = edges_m.shape[1]
    K = A_nodes.shape[-1]

    NPB = NW // B
    CH = -(-N // NPB)
    CH = -(-CH // 128) * 128
    C5 = CH // 128
    NPAD = NPB * CH

    cs_flat = current_state.reshape(B * N, D)
    em_flat = edges_m.reshape(B * E, D)
    cntn = jnp.sum(A_nodes == 0, axis=-1, dtype=jnp.float32).reshape(B * N, 1)
    cnte = jnp.sum(A_edges == 0, axis=-1, dtype=jnp.float32).reshape(B * N, 1)

    def prep_idx(A, stride):
        idx = A.astype(jnp.int32) + (jnp.arange(B, dtype=jnp.int32) * stride)[:, None, None]
        idx_t = idx.transpose(0, 2, 1)
        pad = (jnp.arange(NPAD - N, dtype=jnp.int32) * 1009) % (B * stride)
        pad = jnp.broadcast_to(pad[None, None, :], (B, K, NPAD - N))
        idx_t = jnp.concatenate([idx_t, pad], axis=2)
        idx_t = idx_t.reshape(B, K, NPB, C5, 128).transpose(0, 2, 1, 3, 4)
        return idx_t.reshape(NW, K, C5, 128)

    an_w = prep_idx(A_nodes, N)
    ae_w = prep_idx(A_edges, E)

    ns = _sc_gather_sum(cs_flat, em_flat, an_w, ae_w, B, N, D, K, CH, C5)

    BLK = 2000
    wn = W[:, :D].T
    wc = W[:, D:].T
    ns2 = ns.reshape(B * N, D)
    cs2 = current_state.reshape(B * N, D)
    cs0 = current_state[:, 0:1, :]
    em0 = edges_m[:, 0:1, :]
    nb = N // BLK
    out = pl.pallas_call(
        _mlp_block,
        grid=(B * N // BLK,),
        in_specs=[
            pl.BlockSpec((BLK, D), lambda i: (i, 0)),
            pl.BlockSpec((BLK, D), lambda i: (i, 0)),
            pl.BlockSpec((BLK, 1), lambda i: (i, 0)),
            pl.BlockSpec((BLK, 1), lambda i: (i, 0)),
            pl.BlockSpec((1, 1, D), lambda i: (i // nb, 0, 0)),
            pl.BlockSpec((1, 1, D), lambda i: (i // nb, 0, 0)),
            pl.BlockSpec((D, D), lambda i: (0, 0)),
            pl.BlockSpec((D, D), lambda i: (0, 0)),
            pl.BlockSpec((1, D), lambda i: (0, 0)),
        ],
        out_specs=pl.BlockSpec((1, BLK, D), lambda i: (i // nb, i % nb, 0)),
        out_shape=jax.ShapeDtypeStruct((B, N, D), jnp.float32),
    )(ns2, cs2, cntn, cnte, cs0, em0, wn, wc, b.reshape(1, D))
    return out


# speedup vs baseline: 1.0332x; 1.0009x over previous
import functools

import jax
import jax.numpy as jnp
from jax import lax
from jax.experimental import pallas as pl
from jax.experimental.pallas import tpu as pltpu
from jax.experimental.pallas import tpu_sc as plsc

NC = 2
NS = 16
NW = NC * NS
L = 16


def _sc_gather_sum(cs_flat, em_flat, an_w, ae_w, B, N, D, K, CH, C5):
    NPB = NW // B
    LAST = N - (NPB - 1) * CH

    mesh = plsc.VectorSubcoreMesh(core_axis_name="c", subcore_axis_name="s")

    @functools.partial(
        pl.kernel,
        out_type=jax.ShapeDtypeStruct((B, N, D), jnp.float32),
        mesh=mesh,
        scratch_types=[
            pltpu.VMEM((K, C5, 128), jnp.int32),
            pltpu.VMEM((K, C5, 128), jnp.int32),
            pltpu.VMEM((CH, D), jnp.float32),
            pltpu.SemaphoreType.DMA,
        ],
    )
    def body(cs_ref, em_ref, an_ref, ae_ref, out_ref, idxn_v, idxe_v, acc_v, sem):
        wid = lax.axis_index("s") * NC + lax.axis_index("c")
        b = wid // NPB
        j = wid - b * NPB

        pltpu.sync_copy(an_ref.at[wid], idxn_v)
        pltpu.sync_copy(ae_ref.at[wid], idxe_v)

        init = [
            pltpu.async_copy(
                cs_ref.at[idxn_v.at[0, c]], acc_v.at[pl.ds(c * 128, 128)], sem
            )
            for c in range(C5)
        ]
        for d in init:
            d.wait()

        descs = []
        for g in range(1, K):
            for c in range(C5):
                descs.append(
                    pltpu.async_copy(
                        cs_ref.at[idxn_v.at[g, c]],
                        acc_v.at[pl.ds(c * 128, 128)],
                        sem,
                        add=True,
                    )
                )
        for g in range(K):
            for c in range(C5):
                descs.append(
                    pltpu.async_copy(
                        em_ref.at[idxe_v.at[g, c]],
                        acc_v.at[pl.ds(c * 128, 128)],
                        sem,
                        add=True,
                    )
                )
        for d in descs:
            d.wait()

        row0 = pl.multiple_of(j * CH, 8)

        @pl.when(j == NPB - 1)
        def _():
            pltpu.sync_copy(acc_v.at[pl.ds(0, LAST)], out_ref.at[b, pl.ds(row0, LAST)])

        @pl.when(j != NPB - 1)
        def _():
            pltpu.sync_copy(acc_v, out_ref.at[b, pl.ds(row0, CH)])

    return body(cs_flat, em_flat, an_w, ae_w)


def _mlp_block(ns_ref, cs_ref, cn_ref, ce_ref, cs0_ref, em0_ref, wn_ref, wc_ref,
               b_ref, out_ref):
    ns = ns_ref[...] - cn_ref[...] * cs0_ref[0] - ce_ref[...] * em0_ref[0]
    acc = jnp.dot(ns.astype(jnp.bfloat16), wn_ref[...],
                  preferred_element_type=jnp.float32)
    acc += jnp.dot(cs_ref[...].astype(jnp.bfloat16), wc_ref[...],
                   preferred_element_type=jnp.float32)
    out_ref[0] = jnp.tanh(acc + b_ref[...])


def kernel(current_state, edges_m, A_nodes, A_edges, W, b):
    B, N, D = current_state.shape
    E = edges_m.shape[1]
    K = A_nodes.shape[-1]

    NPB = NW // B
    CH = -(-N // NPB)
    CH = -(-CH // 128) * 128
    C5 = CH // 128
    NPAD = NPB * CH

    cs_flat = current_state.reshape(B * N, D)
    em_flat = edges_m.reshape(B * E, D)
    cntn = jnp.sum(A_nodes == 0, axis=-1, dtype=jnp.float32).reshape(B * N, 1)
    cnte = jnp.sum(A_edges == 0, axis=-1, dtype=jnp.float32).reshape(B * N, 1)

    def prep_idx(A, stride):
        idx = A.astype(jnp.int32) + (jnp.arange(B, dtype=jnp.int32) * stride)[:, None, None]
        idx_t = idx.transpose(0, 2, 1)
        pad = (jnp.arange(NPAD - N, dtype=jnp.int32) * 1009) % (B * stride)
        pad = jnp.broadcast_to(pad[None, None, :], (B, K, NPAD - N))
        idx_t = jnp.concatenate([idx_t, pad], axis=2)
        idx_t = idx_t.reshape(B, K, NPB, C5, 128).transpose(0, 2, 1, 3, 4)
        return idx_t.reshape(NW, K, C5, 128)

    an_w = prep_idx(A_nodes, N)
    ae_w = prep_idx(A_edges, E)

    ns = _sc_gather_sum(cs_flat, em_flat, an_w, ae_w, B, N, D, K, CH, C5)

    BLK = 2000
    wn = W[:, :D].T.astype(jnp.bfloat16)
    wc = W[:, D:].T.astype(jnp.bfloat16)
    ns2 = ns.reshape(B * N, D)
    cs2 = current_state.reshape(B * N, D)
    cs0 = current_state[:, 0:1, :]
    em0 = edges_m[:, 0:1, :]
    nb = N // BLK
    out = pl.pallas_call(
        _mlp_block,
        grid=(B * N // BLK,),
        in_specs=[
            pl.BlockSpec((BLK, D), lambda i: (i, 0)),
            pl.BlockSpec((BLK, D), lambda i: (i, 0)),
            pl.BlockSpec((BLK, 1), lambda i: (i, 0)),
            pl.BlockSpec((BLK, 1), lambda i: (i, 0)),
            pl.BlockSpec((1, 1, D), lambda i: (i // nb, 0, 0)),
            pl.BlockSpec((1, 1, D), lambda i: (i // nb, 0, 0)),
            pl.BlockSpec((D, D), lambda i: (0, 0)),
            pl.BlockSpec((D, D), lambda i: (0, 0)),
            pl.BlockSpec((1, D), lambda i: (0, 0)),
        ],
        out_specs=pl.BlockSpec((1, BLK, D), lambda i: (i // nb, i % nb, 0)),
        out_shape=jax.ShapeDtypeStruct((B, N, D), jnp.float32),
    )(ns2, cs2, cntn, cnte, cs0, em0, wn, wc, b.reshape(1, D))
    return out
